# baseline (speedup 1.0000x reference)
import functools

import jax
import jax.numpy as jnp
from jax.experimental import pallas as pl
from jax.experimental.pallas import tpu as pltpu

_PLANES_PER_STEP = 64


def _make_body(n, d):
    def _body(yyt_ref, out_ref):
        j0 = pl.program_id(0) * _PLANES_PER_STEP
        rolled = pltpu.roll(yyt_ref[:], 2 * n - 1 - j0, axis=1)
        for jj in range(_PLANES_PER_STEP):
            out_ref[jj] = rolled[:, jj : jj + n]

    return _body


def kernel(a, b, c, y):
    n, d = y.shape
    yt = y.T
    yyt = jnp.concatenate([yt, yt], axis=1)
    num_planes = n - 1
    grid = pl.cdiv(num_planes, _PLANES_PER_STEP)
    run = pl.pallas_call(
        _make_body(n, d),
        grid=(grid,),
        in_specs=[pl.BlockSpec((d, 2 * n), lambda g: (0, 0))],
        out_specs=pl.BlockSpec((_PLANES_PER_STEP, d, n), lambda g: (g, 0, 0)),
        out_shape=jax.ShapeDtypeStruct((num_planes, d, n), jnp.float32),
    )
    t = run(yyt)
    return jnp.transpose(t, (2, 0, 1))
